# baseline (speedup 1.0000x reference)
import functools

import jax
import jax.numpy as jnp
from jax import lax
from jax.experimental import pallas as pl
from jax.experimental.pallas import tpu as pltpu
from jax.experimental.pallas import tpu_sc as plsc

N = 10000
E = 320000
D = 128
R = 16
NB = 4

NC = 2
NS = 16
NW = NC * NS
CE = 80
EPW = E // NW
NCH = EPW // CE
SG = 25
NG = NCH // SG
GE = SG * CE
RPT = 632
RPT_LAST = N - 15 * RPT


def _wmix_body(comp_ref, basis_ref, w_ref):
    for r in range(R):
        acc = comp_ref[r, 0] * basis_ref[0]
        for b in range(1, NB):
            acc = acc + comp_ref[r, b] * basis_ref[b]
        w_ref[r] = acc


def _wmix(comp, basis):
    return pl.pallas_call(
        _wmix_body,
        in_specs=[
            pl.BlockSpec(memory_space=pltpu.SMEM),
            pl.BlockSpec(memory_space=pltpu.VMEM),
        ],
        out_specs=pl.BlockSpec(memory_space=pltpu.VMEM),
        out_shape=jax.ShapeDtypeStruct((R, D, D), jnp.float32),
    )(comp, basis)


BN = 2000


def _hrel_body(h_ref, w_ref, out_ref):
    out_ref[0] = jnp.dot(h_ref[...], w_ref[0], preferred_element_type=jnp.float32)


def _hrel(h, w):
    return pl.pallas_call(
        _hrel_body,
        grid=(N // BN, R),
        in_specs=[
            pl.BlockSpec((BN, D), lambda i, r: (i, 0)),
            pl.BlockSpec((1, D, D), lambda i, r: (r, 0, 0)),
        ],
        out_specs=pl.BlockSpec((1, BN, D), lambda i, r: (r, i, 0)),
        out_shape=jax.ShapeDtypeStruct((R, N, D), jnp.float32),
    )(h, w)


def _combine_body(agg_ref, h_ref, wl_ref, b_ref, out_ref, *, relu):
    y = agg_ref[0] + agg_ref[1]
    y = y + jnp.dot(h_ref[...], wl_ref[...], preferred_element_type=jnp.float32)
    y = y + b_ref[...]
    if relu:
        y = jnp.maximum(y, 0.0)
    out_ref[...] = y


def _combine(aggpair, h, w_loop, bias, relu):
    return pl.pallas_call(
        functools.partial(_combine_body, relu=relu),
        grid=(N // BN,),
        in_specs=[
            pl.BlockSpec((2, BN, D), lambda i: (0, i, 0)),
            pl.BlockSpec((BN, D), lambda i: (i, 0)),
            pl.BlockSpec((D, D), lambda i: (0, 0)),
            pl.BlockSpec((1, D), lambda i: (0, 0)),
        ],
        out_specs=pl.BlockSpec((BN, D), lambda i: (i, 0)),
        out_shape=jax.ShapeDtypeStruct((N, D), jnp.float32),
    )(aggpair, h, w_loop, bias)


def _sc_agg_body(ed_hbm, nrm_hbm, hrel_hbm, zeros_hbm, out_hbm,
                 ebuf, nbuf, sidx, dbuf, nsl, rows, agg_sh, sem):
    c = lax.axis_index("c")
    s = lax.axis_index("s")
    w = c * NS + s

    @pl.when(s < NS - 1)
    def _():
        pltpu.sync_copy(zeros_hbm.at[pl.ds(s * RPT, RPT)],
                        agg_sh.at[pl.ds(s * RPT, RPT)])

    @pl.when(s == NS - 1)
    def _():
        pltpu.sync_copy(zeros_hbm.at[pl.ds((NS - 1) * RPT, RPT_LAST)],
                        agg_sh.at[pl.ds((NS - 1) * RPT, RPT_LAST)])

    plsc.subcore_barrier()

    def body(k, carry):
        slot = lax.rem(k, 2)
        pslot = 1 - slot
        gather = pltpu.make_async_copy(
            hrel_hbm.at[sidx.at[slot]], rows.at[slot], sem.at[slot])
        wait_prev = pltpu.make_async_copy(
            hrel_hbm.at[sidx.at[pslot]], rows.at[pslot], sem.at[pslot])

        @pl.when(k < NCH)
        def _():
            @pl.when(lax.rem(k, SG) == 0)
            def _():
                g = w * NG + k // SG
                pltpu.sync_copy(ed_hbm.at[g], ebuf)
                pltpu.sync_copy(nrm_hbm.at[g], nbuf)

            base = lax.rem(k, SG) * CE
            for t in range(CE // 16):
                sl = pl.ds(base + t * 16, 16)
                slo = pl.ds(t * 16, 16)
                sidx[slot, slo] = ebuf[1, sl] * N + ebuf[0, sl]
                dbuf[slot, slo] = ebuf[2, sl]
                nsl[slot, slo] = nbuf[0, sl]
            gather.start()

        @pl.when(k > 0)
        def _():
            wait_prev.wait()

            def edge16(t, c2):
                nvec = nsl[pslot, pl.ds(t * 16, 16)]
                for l in range(16):
                    nv = nvec[l]
                    i = t * 16 + l
                    for j in range(D // 16):
                        sl2 = pl.ds(j * 16, 16)
                        rows[pslot, i, sl2] = rows[pslot, i, sl2] * nv
                return c2
            lax.fori_loop(0, CE // 16, edge16, 0)

            pltpu.sync_copy(rows.at[pslot], agg_sh.at[dbuf.at[pslot]], add=True)
        return carry

    lax.fori_loop(0, NCH + 1, body, 0)
    plsc.subcore_barrier()

    @pl.when(s < NS - 1)
    def _():
        pltpu.sync_copy(agg_sh.at[pl.ds(s * RPT, RPT)],
                        out_hbm.at[pl.ds(c * N + s * RPT, RPT)])

    @pl.when(s == NS - 1)
    def _():
        pltpu.sync_copy(agg_sh.at[pl.ds((NS - 1) * RPT, RPT_LAST)],
                        out_hbm.at[pl.ds(c * N + (NS - 1) * RPT, RPT_LAST)])


def _sc_agg(ed, nrm, hrel_flat, zeros):
    mesh = plsc.VectorSubcoreMesh(core_axis_name="c", subcore_axis_name="s")
    f = functools.partial(
        pl.kernel,
        mesh=mesh,
        out_type=jax.ShapeDtypeStruct((NC * N, D), jnp.float32),
        scratch_types=[
            pltpu.VMEM((3, GE), jnp.int32),
            pltpu.VMEM((1, GE), jnp.float32),
            pltpu.VMEM((2, CE), jnp.int32),
            pltpu.VMEM((2, CE), jnp.int32),
            pltpu.VMEM((2, CE), jnp.float32),
            pltpu.VMEM((2, CE, D), jnp.float32),
            pltpu.VMEM_SHARED((N, D), jnp.float32),
            pltpu.SemaphoreType.DMA((2,)),
        ],
    )(_sc_agg_body)
    return f(ed, nrm, hrel_flat, zeros)


def _layer(h, ed, nrm, zeros, w_basis, w_comp, w_loop, bias, relu):
    w = _wmix(w_comp, w_basis)
    hrel = _hrel(h, w).reshape(R * N, D)
    aggpair = _sc_agg(ed, nrm, hrel, zeros).reshape(NC, N, D)
    return _combine(aggpair, h, w_loop, bias.reshape(1, D), relu)


@jax.jit
def kernel(x, edge_index, edge_type, edge_norm,
           w_basis1, w_comp1, w_loop1, b1,
           w_basis2, w_comp2, w_loop2, b2):
    src = edge_index[0].astype(jnp.int32)
    dst = edge_index[1].astype(jnp.int32)
    ety = edge_type.astype(jnp.int32)
    ed = jnp.stack([src, ety, dst], axis=0)
    ed = ed.reshape(3, NW * NG, GE).transpose(1, 0, 2)
    nrm = edge_norm.astype(jnp.float32).reshape(NW * NG, 1, GE)
    zeros = jnp.zeros((N, D), jnp.float32)

    h1 = _layer(x, ed, nrm, zeros, w_basis1, w_comp1, w_loop1, b1, True)
    out = _layer(h1, ed, nrm, zeros, w_basis2, w_comp2, w_loop2, b2, False)
    return out


# speedup vs baseline: 2.1951x; 2.1951x over previous
import functools

import jax
import jax.numpy as jnp
from jax import lax
from jax.experimental import pallas as pl
from jax.experimental.pallas import tpu as pltpu
from jax.experimental.pallas import tpu_sc as plsc

N = 10000
E = 320000
D = 128
R = 16
NB = 4

NC = 2
NS = 16
NW = NC * NS
CE = 80
EPW = E // NW
NCH = EPW // CE
SG = 25
NG = NCH // SG
GE = SG * CE
RPT = 632
RPT_LAST = N - 15 * RPT


def _wmix_body(comp_ref, basis_ref, w_ref):
    for r in range(R):
        acc = comp_ref[r, 0] * basis_ref[0]
        for b in range(1, NB):
            acc = acc + comp_ref[r, b] * basis_ref[b]
        w_ref[r] = acc


def _wmix(comp, basis):
    return pl.pallas_call(
        _wmix_body,
        in_specs=[
            pl.BlockSpec(memory_space=pltpu.SMEM),
            pl.BlockSpec(memory_space=pltpu.VMEM),
        ],
        out_specs=pl.BlockSpec(memory_space=pltpu.VMEM),
        out_shape=jax.ShapeDtypeStruct((R, D, D), jnp.float32),
    )(comp, basis)


BN = 2000


def _hrel_body(h_ref, w_ref, out_ref):
    out_ref[0] = jnp.dot(h_ref[...], w_ref[0], preferred_element_type=jnp.float32)


def _hrel(h, w):
    return pl.pallas_call(
        _hrel_body,
        grid=(N // BN, R),
        in_specs=[
            pl.BlockSpec((BN, D), lambda i, r: (i, 0)),
            pl.BlockSpec((1, D, D), lambda i, r: (r, 0, 0)),
        ],
        out_specs=pl.BlockSpec((1, BN, D), lambda i, r: (r, i, 0)),
        out_shape=jax.ShapeDtypeStruct((R, N, D), jnp.float32),
    )(h, w)


def _combine_body(agg_ref, h_ref, wl_ref, b_ref, out_ref, *, relu):
    y = agg_ref[0] + agg_ref[1]
    y = y + jnp.dot(h_ref[...], wl_ref[...], preferred_element_type=jnp.float32)
    y = y + b_ref[...]
    if relu:
        y = jnp.maximum(y, 0.0)
    out_ref[...] = y


def _combine(aggpair, h, w_loop, bias, relu):
    return pl.pallas_call(
        functools.partial(_combine_body, relu=relu),
        grid=(N // BN,),
        in_specs=[
            pl.BlockSpec((2, BN, D), lambda i: (0, i, 0)),
            pl.BlockSpec((BN, D), lambda i: (i, 0)),
            pl.BlockSpec((D, D), lambda i: (0, 0)),
            pl.BlockSpec((1, D), lambda i: (0, 0)),
        ],
        out_specs=pl.BlockSpec((BN, D), lambda i: (i, 0)),
        out_shape=jax.ShapeDtypeStruct((N, D), jnp.float32),
    )(aggpair, h, w_loop, bias)


def _sc_agg_body(ed_hbm, nrm_hbm, hrel_hbm, zeros_hbm, out_hbm,
                 ebuf, nbuf, sidx, dbuf, nsl, rows, agg_sh, sem):
    c = lax.axis_index("c")
    s = lax.axis_index("s")
    w = c * NS + s

    @pl.when(s < NS - 1)
    def _():
        pltpu.sync_copy(zeros_hbm.at[pl.ds(s * RPT, RPT)],
                        agg_sh.at[pl.ds(s * RPT, RPT)])

    @pl.when(s == NS - 1)
    def _():
        pltpu.sync_copy(zeros_hbm.at[pl.ds((NS - 1) * RPT, RPT_LAST)],
                        agg_sh.at[pl.ds((NS - 1) * RPT, RPT_LAST)])

    plsc.subcore_barrier()

    def unpack(k, slot):
        @pl.when(lax.rem(k, SG) == 0)
        def _():
            g = w * NG + k // SG
            pltpu.sync_copy(ed_hbm.at[g], ebuf)
            pltpu.sync_copy(nrm_hbm.at[g], nbuf)

        base = lax.rem(k, SG) * CE
        for t in range(CE // 16):
            sl = pl.ds(base + t * 16, 16)
            slo = pl.ds(t * 16, 16)
            sidx[slot, slo] = ebuf[1, sl] * N + ebuf[0, sl]
            dbuf[slot, slo] = ebuf[2, sl]
            nsl[slot, slo] = nbuf[0, sl]

    def gather(slot):
        return pltpu.make_async_copy(
            hrel_hbm.at[sidx.at[slot]], rows.at[slot], sem.at[slot])

    def process(slot):
        def edge16(t, c2):
            nvec = nsl[slot, pl.ds(t * 16, 16)]
            for l in range(16):
                nv = nvec[l]
                i = t * 16 + l
                for j in range(D // 16):
                    sl2 = pl.ds(j * 16, 16)
                    rows[slot, i, sl2] = rows[slot, i, sl2] * nv
            return c2
        lax.fori_loop(0, CE // 16, edge16, 0)
        pltpu.sync_copy(rows.at[slot], agg_sh.at[dbuf.at[slot]], add=True)

    unpack(0, 0)
    gather(0).start()

    def body(m, carry):
        k0 = 2 * m
        unpack(k0 + 1, 1)
        gather(1).start()
        gather(0).wait()
        process(0)
        unpack(k0 + 2, 0)
        gather(0).start()
        gather(1).wait()
        process(1)
        return carry

    lax.fori_loop(0, (NCH - 1) // 2, body, 0)
    gather(0).wait()
    process(0)
    plsc.subcore_barrier()

    @pl.when(s < NS - 1)
    def _():
        pltpu.sync_copy(agg_sh.at[pl.ds(s * RPT, RPT)],
                        out_hbm.at[pl.ds(c * N + s * RPT, RPT)])

    @pl.when(s == NS - 1)
    def _():
        pltpu.sync_copy(agg_sh.at[pl.ds((NS - 1) * RPT, RPT_LAST)],
                        out_hbm.at[pl.ds(c * N + (NS - 1) * RPT, RPT_LAST)])


def _sc_agg(ed, nrm, hrel_flat, zeros):
    mesh = plsc.VectorSubcoreMesh(core_axis_name="c", subcore_axis_name="s")
    f = functools.partial(
        pl.kernel,
        mesh=mesh,
        out_type=jax.ShapeDtypeStruct((NC * N, D), jnp.float32),
        scratch_types=[
            pltpu.VMEM((3, GE), jnp.int32),
            pltpu.VMEM((1, GE), jnp.float32),
            pltpu.VMEM((2, CE), jnp.int32),
            pltpu.VMEM((2, CE), jnp.int32),
            pltpu.VMEM((2, CE), jnp.float32),
            pltpu.VMEM((2, CE, D), jnp.float32),
            pltpu.VMEM_SHARED((N, D), jnp.float32),
            pltpu.SemaphoreType.DMA((2,)),
        ],
    )(_sc_agg_body)
    return f(ed, nrm, hrel_flat, zeros)


def _layer(h, ed, nrm, zeros, w_basis, w_comp, w_loop, bias, relu):
    w = _wmix(w_comp, w_basis)
    hrel = _hrel(h, w).reshape(R * N, D)
    aggpair = _sc_agg(ed, nrm, hrel, zeros).reshape(NC, N, D)
    return _combine(aggpair, h, w_loop, bias.reshape(1, D), relu)


@jax.jit
def kernel(x, edge_index, edge_type, edge_norm,
           w_basis1, w_comp1, w_loop1, b1,
           w_basis2, w_comp2, w_loop2, b2):
    src = edge_index[0].astype(jnp.int32)
    dst = edge_index[1].astype(jnp.int32)
    ety = edge_type.astype(jnp.int32)
    ed = jnp.stack([src, ety, dst], axis=0)
    ed = ed.reshape(3, NW * NG, GE).transpose(1, 0, 2)
    nrm = edge_norm.astype(jnp.float32).reshape(NW * NG, 1, GE)
    zeros = jnp.zeros((N, D), jnp.float32)

    h1 = _layer(x, ed, nrm, zeros, w_basis1, w_comp1, w_loop1, b1, True)
    out = _layer(h1, ed, nrm, zeros, w_basis2, w_comp2, w_loop2, b2, False)
    return out


# speedup vs baseline: 2.6748x; 1.2185x over previous
import functools

import jax
import jax.numpy as jnp
from jax import lax
from jax.experimental import pallas as pl
from jax.experimental.pallas import tpu as pltpu
from jax.experimental.pallas import tpu_sc as plsc

N = 10000
E = 320000
D = 128
R = 16
NB = 4

NC = 2
NS = 16
NW = NC * NS
CE = 80
EPW = E // NW
NCH = EPW // CE
SG = 25
NG = NCH // SG
GE = SG * CE
RPT = 632
RPT_LAST = N - 15 * RPT


BN = 2000


def _mix_w(comp_ref, basis_ref, w_ref):
    for r in range(R):
        acc = comp_ref[r, 0] * basis_ref[0]
        for b in range(1, NB):
            acc = acc + comp_ref[r, b] * basis_ref[b]
        w_ref[r] = acc


def _hrel_body(comp_ref, basis_ref, h_ref, out_ref, w_ref):
    @pl.when(jnp.logical_and(pl.program_id(0) == 0, pl.program_id(1) == 0))
    def _():
        _mix_w(comp_ref, basis_ref, w_ref)

    r = pl.program_id(1)
    out_ref[0] = jnp.dot(h_ref[...], w_ref[r],
                         preferred_element_type=jnp.float32)


def _hrel(h, comp, basis):
    return pl.pallas_call(
        _hrel_body,
        grid=(N // BN, R),
        in_specs=[
            pl.BlockSpec(memory_space=pltpu.SMEM),
            pl.BlockSpec((NB, D, D), lambda i, r: (0, 0, 0)),
            pl.BlockSpec((BN, D), lambda i, r: (i, 0)),
        ],
        out_specs=pl.BlockSpec((1, BN, D), lambda i, r: (r, i, 0)),
        out_shape=jax.ShapeDtypeStruct((R, N, D), jnp.float32),
        scratch_shapes=[pltpu.VMEM((R, D, D), jnp.float32)],
    )(comp, basis, h)


def _mid_body(agg_ref, h_ref, wl_ref, b_ref, comp_ref, basis_ref,
              h1_ref, out2_ref, w_ref):
    @pl.when(pl.program_id(0) == 0)
    def _():
        _mix_w(comp_ref, basis_ref, w_ref)

    y = agg_ref[0] + agg_ref[1]
    y = y + jnp.dot(h_ref[...], wl_ref[...], preferred_element_type=jnp.float32)
    y = jnp.maximum(y + b_ref[...], 0.0)
    h1_ref[...] = y
    for r in range(R):
        out2_ref[r] = jnp.dot(y, w_ref[r], preferred_element_type=jnp.float32)


def _mid(aggpair, h, w_loop, bias, comp2, basis2):
    return pl.pallas_call(
        _mid_body,
        grid=(N // BN,),
        in_specs=[
            pl.BlockSpec((2, BN, D), lambda i: (0, i, 0)),
            pl.BlockSpec((BN, D), lambda i: (i, 0)),
            pl.BlockSpec((D, D), lambda i: (0, 0)),
            pl.BlockSpec((1, D), lambda i: (0, 0)),
            pl.BlockSpec(memory_space=pltpu.SMEM),
            pl.BlockSpec((NB, D, D), lambda i: (0, 0, 0)),
        ],
        out_specs=[
            pl.BlockSpec((BN, D), lambda i: (i, 0)),
            pl.BlockSpec((R, BN, D), lambda i: (0, i, 0)),
        ],
        out_shape=[
            jax.ShapeDtypeStruct((N, D), jnp.float32),
            jax.ShapeDtypeStruct((R, N, D), jnp.float32),
        ],
        scratch_shapes=[pltpu.VMEM((R, D, D), jnp.float32)],
    )(aggpair, h, w_loop, bias, comp2, basis2)


def _combine_body(agg_ref, h_ref, wl_ref, b_ref, out_ref):
    y = agg_ref[0] + agg_ref[1]
    y = y + jnp.dot(h_ref[...], wl_ref[...], preferred_element_type=jnp.float32)
    out_ref[...] = y + b_ref[...]


def _combine(aggpair, h, w_loop, bias):
    return pl.pallas_call(
        _combine_body,
        grid=(N // BN,),
        in_specs=[
            pl.BlockSpec((2, BN, D), lambda i: (0, i, 0)),
            pl.BlockSpec((BN, D), lambda i: (i, 0)),
            pl.BlockSpec((D, D), lambda i: (0, 0)),
            pl.BlockSpec((1, D), lambda i: (0, 0)),
        ],
        out_specs=pl.BlockSpec((BN, D), lambda i: (i, 0)),
        out_shape=jax.ShapeDtypeStruct((N, D), jnp.float32),
    )(aggpair, h, w_loop, bias)


def _sc_agg_body(ed_hbm, nrm_hbm, hrel_hbm, zeros_hbm, out_hbm,
                 ebuf, nbuf, sidx, dbuf, nsl, rows, agg_sh, sem):
    c = lax.axis_index("c")
    s = lax.axis_index("s")
    w = c * NS + s

    @pl.when(s < NS - 1)
    def _():
        pltpu.sync_copy(zeros_hbm.at[pl.ds(s * RPT, RPT)],
                        agg_sh.at[pl.ds(s * RPT, RPT)])

    @pl.when(s == NS - 1)
    def _():
        pltpu.sync_copy(zeros_hbm.at[pl.ds((NS - 1) * RPT, RPT_LAST)],
                        agg_sh.at[pl.ds((NS - 1) * RPT, RPT_LAST)])

    plsc.subcore_barrier()

    def unpack(k, slot):
        @pl.when(lax.rem(k, SG) == 0)
        def _():
            g = w * NG + k // SG
            pltpu.sync_copy(ed_hbm.at[g], ebuf)
            pltpu.sync_copy(nrm_hbm.at[g], nbuf)

        base = lax.rem(k, SG) * CE
        for t in range(CE // 16):
            sl = pl.ds(base + t * 16, 16)
            slo = pl.ds(t * 16, 16)
            sidx[slot, slo] = ebuf[1, sl] * N + ebuf[0, sl]
            dbuf[slot, slo] = ebuf[2, sl]
            nsl[slot, slo] = nbuf[0, sl]

    def gather(slot):
        return pltpu.make_async_copy(
            hrel_hbm.at[sidx.at[slot]], rows.at[slot], sem.at[slot])

    def scatter(slot):
        return pltpu.make_async_copy(
            rows.at[slot], agg_sh.at[dbuf.at[slot]], sem.at[3 + slot])

    def scale(slot):
        def edge16(t, c2):
            nvec = nsl[slot, pl.ds(t * 16, 16)]
            for l in range(16):
                nv = nvec[l]
                i = t * 16 + l
                for j in range(D // 16):
                    sl2 = pl.ds(j * 16, 16)
                    rows[slot, i, sl2] = rows[slot, i, sl2] * nv
            return c2
        lax.fori_loop(0, CE // 16, edge16, 0)

    def prep(kp, p):
        @pl.when(kp < NCH)
        def _():
            @pl.when(kp >= 3)
            def _():
                scatter(p).wait()
            unpack(kp, p)
            gather(p).start()

    def leg(k, q):
        gather(q).wait()
        scale(q)
        scatter(q).start(add=True)
        prep(k + 2, (q + 2) % 3)

    unpack(0, 0)
    gather(0).start()
    unpack(1, 1)
    gather(1).start()

    def body(m, carry):
        k0 = 3 * m
        leg(k0, 0)
        leg(k0 + 1, 1)
        leg(k0 + 2, 2)
        return carry

    lax.fori_loop(0, NCH // 3, body, 0)
    gather(0).wait()
    scale(0)
    scatter(0).start(add=True)
    gather(1).wait()
    scale(1)
    scatter(1).start(add=True)
    scatter(2).wait()
    scatter(0).wait()
    scatter(1).wait()
    plsc.subcore_barrier()

    @pl.when(s < NS - 1)
    def _():
        pltpu.sync_copy(agg_sh.at[pl.ds(s * RPT, RPT)],
                        out_hbm.at[pl.ds(c * N + s * RPT, RPT)])

    @pl.when(s == NS - 1)
    def _():
        pltpu.sync_copy(agg_sh.at[pl.ds((NS - 1) * RPT, RPT_LAST)],
                        out_hbm.at[pl.ds(c * N + (NS - 1) * RPT, RPT_LAST)])


def _sc_agg(ed, nrm, hrel_flat, zeros):
    mesh = plsc.VectorSubcoreMesh(core_axis_name="c", subcore_axis_name="s")
    f = functools.partial(
        pl.kernel,
        mesh=mesh,
        out_type=jax.ShapeDtypeStruct((NC * N, D), jnp.float32),
        scratch_types=[
            pltpu.VMEM((3, GE), jnp.int32),
            pltpu.VMEM((1, GE), jnp.float32),
            pltpu.VMEM((3, CE), jnp.int32),
            pltpu.VMEM((3, CE), jnp.int32),
            pltpu.VMEM((3, CE), jnp.float32),
            pltpu.VMEM((3, CE, D), jnp.float32),
            pltpu.VMEM_SHARED((N, D), jnp.float32),
            pltpu.SemaphoreType.DMA((6,)),
        ],
    )(_sc_agg_body)
    return f(ed, nrm, hrel_flat, zeros)


@jax.jit
def kernel(x, edge_index, edge_type, edge_norm,
           w_basis1, w_comp1, w_loop1, b1,
           w_basis2, w_comp2, w_loop2, b2):
    src = edge_index[0].astype(jnp.int32)
    dst = edge_index[1].astype(jnp.int32)
    ety = edge_type.astype(jnp.int32)
    ed = jnp.stack([src, ety, dst], axis=0)
    ed = ed.reshape(3, NW * NG, GE).transpose(1, 0, 2)
    nrm = edge_norm.astype(jnp.float32).reshape(NW * NG, 1, GE)
    zeros = jnp.zeros((N, D), jnp.float32)

    hrel1 = _hrel(x, w_comp1, w_basis1).reshape(R * N, D)
    agg1 = _sc_agg(ed, nrm, hrel1, zeros).reshape(NC, N, D)
    h1, hrel2 = _mid(agg1, x, w_loop1, b1.reshape(1, D), w_comp2, w_basis2)
    agg2 = _sc_agg(ed, nrm, hrel2.reshape(R * N, D), zeros).reshape(NC, N, D)
    return _combine(agg2, h1, w_loop2, b2.reshape(1, D))


# speedup vs baseline: 2.6990x; 1.0090x over previous
import functools

import jax
import jax.numpy as jnp
from jax import lax
from jax.experimental import pallas as pl
from jax.experimental.pallas import tpu as pltpu
from jax.experimental.pallas import tpu_sc as plsc

N = 10000
E = 320000
D = 128
R = 16
NB = 4

NC = 2
NS = 16
NW = NC * NS
CE = 80
EPW = E // NW
NCH = EPW // CE
SG = 25
NG = NCH // SG
GE = SG * CE
RPT = 632
RPT_LAST = N - 15 * RPT


BN = 2000


def _mix_w(comp_ref, basis_ref, w_ref):
    for r in range(R):
        acc = comp_ref[r, 0] * basis_ref[0]
        for b in range(1, NB):
            acc = acc + comp_ref[r, b] * basis_ref[b]
        w_ref[r] = acc


def _hrel_body(comp_ref, basis_ref, h_ref, out_ref, w_ref):
    @pl.when(jnp.logical_and(pl.program_id(0) == 0, pl.program_id(1) == 0))
    def _():
        _mix_w(comp_ref, basis_ref, w_ref)

    r = pl.program_id(1)
    out_ref[0] = jnp.dot(h_ref[...], w_ref[r],
                         preferred_element_type=jnp.float32)


def _hrel(h, comp, basis):
    return pl.pallas_call(
        _hrel_body,
        grid=(N // BN, R),
        in_specs=[
            pl.BlockSpec(memory_space=pltpu.SMEM),
            pl.BlockSpec((NB, D, D), lambda i, r: (0, 0, 0)),
            pl.BlockSpec((BN, D), lambda i, r: (i, 0)),
        ],
        out_specs=pl.BlockSpec((1, BN, D), lambda i, r: (r, i, 0)),
        out_shape=jax.ShapeDtypeStruct((R, N, D), jnp.float32),
        scratch_shapes=[pltpu.VMEM((R, D, D), jnp.float32)],
    )(comp, basis, h)


def _mid_body(agg_ref, h_ref, wl_ref, b_ref, comp_ref, basis_ref,
              h1_ref, out2_ref, w_ref):
    @pl.when(pl.program_id(0) == 0)
    def _():
        _mix_w(comp_ref, basis_ref, w_ref)

    y = agg_ref[0] + agg_ref[1]
    y = y + jnp.dot(h_ref[...], wl_ref[...], preferred_element_type=jnp.float32)
    y = jnp.maximum(y + b_ref[...], 0.0)
    h1_ref[...] = y
    for r in range(R):
        out2_ref[r] = jnp.dot(y, w_ref[r], preferred_element_type=jnp.float32)


def _mid(aggpair, h, w_loop, bias, comp2, basis2):
    return pl.pallas_call(
        _mid_body,
        grid=(N // BN,),
        in_specs=[
            pl.BlockSpec((2, BN, D), lambda i: (0, i, 0)),
            pl.BlockSpec((BN, D), lambda i: (i, 0)),
            pl.BlockSpec((D, D), lambda i: (0, 0)),
            pl.BlockSpec((1, D), lambda i: (0, 0)),
            pl.BlockSpec(memory_space=pltpu.SMEM),
            pl.BlockSpec((NB, D, D), lambda i: (0, 0, 0)),
        ],
        out_specs=[
            pl.BlockSpec((BN, D), lambda i: (i, 0)),
            pl.BlockSpec((R, BN, D), lambda i: (0, i, 0)),
        ],
        out_shape=[
            jax.ShapeDtypeStruct((N, D), jnp.float32),
            jax.ShapeDtypeStruct((R, N, D), jnp.float32),
        ],
        scratch_shapes=[pltpu.VMEM((R, D, D), jnp.float32)],
    )(aggpair, h, w_loop, bias, comp2, basis2)


def _combine_body(agg_ref, h_ref, wl_ref, b_ref, out_ref):
    y = agg_ref[0] + agg_ref[1]
    y = y + jnp.dot(h_ref[...], wl_ref[...], preferred_element_type=jnp.float32)
    out_ref[...] = y + b_ref[...]


def _combine(aggpair, h, w_loop, bias):
    return pl.pallas_call(
        _combine_body,
        grid=(N // BN,),
        in_specs=[
            pl.BlockSpec((2, BN, D), lambda i: (0, i, 0)),
            pl.BlockSpec((BN, D), lambda i: (i, 0)),
            pl.BlockSpec((D, D), lambda i: (0, 0)),
            pl.BlockSpec((1, D), lambda i: (0, 0)),
        ],
        out_specs=pl.BlockSpec((BN, D), lambda i: (i, 0)),
        out_shape=jax.ShapeDtypeStruct((N, D), jnp.float32),
    )(aggpair, h, w_loop, bias)


def _sc_agg_body(src_hbm, ety_hbm, dst_hbm, nrm_hbm, hrel_hbm, zeros_hbm,
                 out_hbm, sbuf, tbuf, qbuf, nbuf, sidx, dbuf, nsl, rows,
                 agg_sh, sem):
    c = lax.axis_index("c")
    s = lax.axis_index("s")
    w = c * NS + s

    @pl.when(s < NS - 1)
    def _():
        pltpu.sync_copy(zeros_hbm.at[pl.ds(s * RPT, RPT)],
                        agg_sh.at[pl.ds(s * RPT, RPT)])

    @pl.when(s == NS - 1)
    def _():
        pltpu.sync_copy(zeros_hbm.at[pl.ds((NS - 1) * RPT, RPT_LAST)],
                        agg_sh.at[pl.ds((NS - 1) * RPT, RPT_LAST)])

    plsc.subcore_barrier()

    def unpack(k, slot):
        @pl.when(lax.rem(k, SG) == 0)
        def _():
            gbase = w * EPW + (k // SG) * GE
            pltpu.sync_copy(src_hbm.at[pl.ds(gbase, GE)], sbuf)
            pltpu.sync_copy(ety_hbm.at[pl.ds(gbase, GE)], tbuf)
            pltpu.sync_copy(dst_hbm.at[pl.ds(gbase, GE)], qbuf)
            pltpu.sync_copy(nrm_hbm.at[pl.ds(gbase, GE)], nbuf)

        base = lax.rem(k, SG) * CE
        for t in range(CE // 16):
            sl = pl.ds(base + t * 16, 16)
            slo = pl.ds(t * 16, 16)
            sidx[slot, slo] = tbuf[sl] * N + sbuf[sl]
            dbuf[slot, slo] = qbuf[sl]
            nsl[slot, slo] = nbuf[sl]

    def gather(slot):
        return pltpu.make_async_copy(
            hrel_hbm.at[sidx.at[slot]], rows.at[slot], sem.at[slot])

    def scatter(slot):
        return pltpu.make_async_copy(
            rows.at[slot], agg_sh.at[dbuf.at[slot]], sem.at[3 + slot])

    def scale(slot):
        def edge16(t, c2):
            nvec = nsl[slot, pl.ds(t * 16, 16)]
            for l in range(16):
                nv = nvec[l]
                i = t * 16 + l
                for j in range(D // 16):
                    sl2 = pl.ds(j * 16, 16)
                    rows[slot, i, sl2] = rows[slot, i, sl2] * nv
            return c2
        lax.fori_loop(0, CE // 16, edge16, 0)

    def prep(kp, p):
        @pl.when(kp < NCH)
        def _():
            @pl.when(kp >= 3)
            def _():
                scatter(p).wait()
            unpack(kp, p)
            gather(p).start()

    def leg(k, q):
        gather(q).wait()
        scale(q)
        scatter(q).start(add=True)
        prep(k + 2, (q + 2) % 3)

    unpack(0, 0)
    gather(0).start()
    unpack(1, 1)
    gather(1).start()

    def body(m, carry):
        k0 = 3 * m
        leg(k0, 0)
        leg(k0 + 1, 1)
        leg(k0 + 2, 2)
        return carry

    lax.fori_loop(0, NCH // 3, body, 0)
    gather(0).wait()
    scale(0)
    scatter(0).start(add=True)
    gather(1).wait()
    scale(1)
    scatter(1).start(add=True)
    scatter(2).wait()
    scatter(0).wait()
    scatter(1).wait()
    plsc.subcore_barrier()

    @pl.when(s < NS - 1)
    def _():
        pltpu.sync_copy(agg_sh.at[pl.ds(s * RPT, RPT)],
                        out_hbm.at[pl.ds(c * N + s * RPT, RPT)])

    @pl.when(s == NS - 1)
    def _():
        pltpu.sync_copy(agg_sh.at[pl.ds((NS - 1) * RPT, RPT_LAST)],
                        out_hbm.at[pl.ds(c * N + (NS - 1) * RPT, RPT_LAST)])


def _sc_agg(src, ety, dst, nrm, hrel_flat, zeros):
    mesh = plsc.VectorSubcoreMesh(core_axis_name="c", subcore_axis_name="s")
    f = functools.partial(
        pl.kernel,
        mesh=mesh,
        out_type=jax.ShapeDtypeStruct((NC * N, D), jnp.float32),
        scratch_types=[
            pltpu.VMEM((GE,), jnp.int32),
            pltpu.VMEM((GE,), jnp.int32),
            pltpu.VMEM((GE,), jnp.int32),
            pltpu.VMEM((GE,), jnp.float32),
            pltpu.VMEM((3, CE), jnp.int32),
            pltpu.VMEM((3, CE), jnp.int32),
            pltpu.VMEM((3, CE), jnp.float32),
            pltpu.VMEM((3, CE, D), jnp.float32),
            pltpu.VMEM_SHARED((N, D), jnp.float32),
            pltpu.SemaphoreType.DMA((6,)),
        ],
    )(_sc_agg_body)
    return f(src, ety, dst, nrm, hrel_flat, zeros)


@jax.jit
def kernel(x, edge_index, edge_type, edge_norm,
           w_basis1, w_comp1, w_loop1, b1,
           w_basis2, w_comp2, w_loop2, b2):
    src = edge_index[0].astype(jnp.int32)
    dst = edge_index[1].astype(jnp.int32)
    ety = edge_type.astype(jnp.int32)
    nrm = edge_norm.astype(jnp.float32)
    zeros = jnp.zeros((N, D), jnp.float32)

    hrel1 = _hrel(x, w_comp1, w_basis1).reshape(R * N, D)
    agg1 = _sc_agg(src, ety, dst, nrm, hrel1, zeros).reshape(NC, N, D)
    h1, hrel2 = _mid(agg1, x, w_loop1, b1.reshape(1, D), w_comp2, w_basis2)
    agg2 = _sc_agg(src, ety, dst, nrm, hrel2.reshape(R * N, D), zeros).reshape(NC, N, D)
    return _combine(agg2, h1, w_loop2, b2.reshape(1, D))
